# baseline (device time: 155120 ns/iter reference)
import jax
import jax.numpy as jnp
from jax import lax
from jax.experimental import pallas as pl
from jax.experimental.pallas import tpu as pltpu

N_DEV = 8
B_LOC = 2
SQ = 128
D = 512
H_LOC = 8
DH = 64
SCALE = 0.125


def kernel(x, Wq, Wo, Wk, Wv):
    def body(x_ref, wq_ref, wo_ref, wk_ref, wv_ref, out_ref,
             x_all, acc, rs_recv, o_buf,
             ag_send_sems, ag_recv_sems, rs_send_sems, rs_recv_sems):
        my = lax.axis_index("i")
        left = lax.rem(my + (N_DEV - 1), N_DEV)
        right = lax.rem(my + 1, N_DEV)

        barrier_sem = pltpu.get_barrier_semaphore()
        pl.semaphore_signal(barrier_sem, inc=1, device_id=(left,),
                            device_id_type=pl.DeviceIdType.MESH)
        pl.semaphore_signal(barrier_sem, inc=1, device_id=(right,),
                            device_id_type=pl.DeviceIdType.MESH)
        pl.semaphore_wait(barrier_sem, 2)

        x_all[my] = x_ref[...]

        for h in range(N_DEV - 1):
            slot = lax.rem(my + (N_DEV - h), N_DEV)
            rdma = pltpu.make_async_remote_copy(
                src_ref=x_all.at[slot],
                dst_ref=x_all.at[slot],
                send_sem=ag_send_sems.at[h],
                recv_sem=ag_recv_sems.at[h],
                device_id=(right,),
                device_id_type=pl.DeviceIdType.MESH,
            )
            rdma.start()
            rdma.wait()

        for j in range(N_DEV):
            for b in range(B_LOC):
                xb = x_all[j, b]
                q = jnp.dot(xb, wq_ref[...],
                            preferred_element_type=jnp.float32)
                k = jnp.dot(xb, wk_ref[...],
                            preferred_element_type=jnp.float32)
                v = jnp.dot(xb, wv_ref[...],
                            preferred_element_type=jnp.float32)
                for hh in range(H_LOC):
                    sl = slice(hh * DH, (hh + 1) * DH)
                    s = lax.dot_general(
                        q[:, sl], k[:, sl],
                        (((1,), (1,)), ((), ())),
                        preferred_element_type=jnp.float32,
                    ) * SCALE
                    m = jnp.max(s, axis=1, keepdims=True)
                    p = jnp.exp(s - m)
                    l = jnp.sum(p, axis=1, keepdims=True)
                    o = jnp.dot(p, v[:, sl],
                                preferred_element_type=jnp.float32)
                    o_buf[:, sl] = o / l
                acc[j, b] = jnp.dot(o_buf[...], wo_ref[...],
                                    preferred_element_type=jnp.float32)

        for t in range(N_DEV - 1):
            c_send = lax.rem(my + (N_DEV - t - 1), N_DEV)
            rdma = pltpu.make_async_remote_copy(
                src_ref=acc.at[c_send],
                dst_ref=rs_recv.at[t],
                send_sem=rs_send_sems.at[t],
                recv_sem=rs_recv_sems.at[t],
                device_id=(right,),
                device_id_type=pl.DeviceIdType.MESH,
            )
            rdma.start()
            rdma.wait()
            c_recv = lax.rem(my + (N_DEV - t - 2), N_DEV)
            acc[c_recv] = acc[c_recv] + rs_recv[t]

        out_ref[...] = acc[my]

    return pl.pallas_call(
        body,
        out_shape=jax.ShapeDtypeStruct((B_LOC, SQ, D), jnp.float32),
        in_specs=[pl.BlockSpec(memory_space=pltpu.VMEM)] * 5,
        out_specs=pl.BlockSpec(memory_space=pltpu.VMEM),
        scratch_shapes=[
            pltpu.VMEM((N_DEV, B_LOC, SQ, D), jnp.float32),
            pltpu.VMEM((N_DEV, B_LOC, SQ, D), jnp.float32),
            pltpu.VMEM((N_DEV - 1, B_LOC, SQ, D), jnp.float32),
            pltpu.VMEM((SQ, D), jnp.float32),
            pltpu.SemaphoreType.DMA((N_DEV - 1,)),
            pltpu.SemaphoreType.DMA((N_DEV - 1,)),
            pltpu.SemaphoreType.DMA((N_DEV - 1,)),
            pltpu.SemaphoreType.DMA((N_DEV - 1,)),
        ],
        compiler_params=pltpu.CompilerParams(collective_id=0),
    )(x, Wq, Wo, Wk, Wv)


# device time: 65435 ns/iter; 2.3706x vs baseline; 2.3706x over previous
import jax
import jax.numpy as jnp
from jax import lax
from jax.experimental import pallas as pl
from jax.experimental.pallas import tpu as pltpu

N_DEV = 8
B_LOC = 2
SQ = 128
D = 512
H_LOC = 8
DH = 64
SCALE = 0.125


def kernel(x, Wq, Wo, Wk, Wv):
    def body(x_ref, wq_ref, wo_ref, wk_ref, wv_ref, out_ref,
             x_r, x_l, acc_r, acc_l, rsbuf_r, rsbuf_l, o_buf,
             ag_r_send, ag_r_recv, ag_l_send, ag_l_recv,
             rs_r_send, rs_r_recv, rs_l_send, rs_l_recv):
        my = lax.axis_index("i")
        left = lax.rem(my + (N_DEV - 1), N_DEV)
        right = lax.rem(my + 1, N_DEV)

        def compute_panel(x_buf, slot):
            xb = x_buf[slot]
            q = jnp.dot(xb, wq_ref[...], preferred_element_type=jnp.float32)
            k = jnp.dot(xb, wk_ref[...], preferred_element_type=jnp.float32)
            v = jnp.dot(xb, wv_ref[...], preferred_element_type=jnp.float32)
            for hh in range(H_LOC):
                sl = slice(hh * DH, (hh + 1) * DH)
                s = lax.dot_general(
                    q[:, sl], k[:, sl], (((1,), (1,)), ((), ())),
                    preferred_element_type=jnp.float32,
                ) * SCALE
                m = jnp.max(s, axis=1, keepdims=True)
                p = jnp.exp(s - m)
                l = jnp.sum(p, axis=1, keepdims=True)
                o = jnp.dot(p, v[:, sl], preferred_element_type=jnp.float32)
                o_buf[:, sl] = o / l
            return jnp.dot(o_buf[...], wo_ref[...],
                           preferred_element_type=jnp.float32)

        def ag_copy(x_buf, slot, sems, h, dst):
            return pltpu.make_async_remote_copy(
                src_ref=x_buf.at[slot], dst_ref=x_buf.at[slot],
                send_sem=sems[0].at[h], recv_sem=sems[1].at[h],
                device_id=(dst,), device_id_type=pl.DeviceIdType.MESH,
            )

        def rs_copy(acc_buf, slot, rsbuf, sems, t, dst):
            return pltpu.make_async_remote_copy(
                src_ref=acc_buf.at[slot], dst_ref=rsbuf.at[t],
                send_sem=sems[0].at[t], recv_sem=sems[1].at[t],
                device_id=(dst,), device_id_type=pl.DeviceIdType.MESH,
            )

        barrier_sem = pltpu.get_barrier_semaphore()
        pl.semaphore_signal(barrier_sem, inc=1, device_id=(left,),
                            device_id_type=pl.DeviceIdType.MESH)
        pl.semaphore_signal(barrier_sem, inc=1, device_id=(right,),
                            device_id_type=pl.DeviceIdType.MESH)
        pl.semaphore_wait(barrier_sem, 2)

        x_r[my] = x_ref[0]
        x_l[my] = x_ref[1]
        pending = []
        ag_r = ag_copy(x_r, my, (ag_r_send, ag_r_recv), 0, right)
        ag_r.start()
        ag_l = ag_copy(x_l, my, (ag_l_send, ag_l_recv), 0, left)
        ag_l.start()
        pending += [ag_r, ag_l]

        acc_r[my] = compute_panel(x_r, my)
        acc_l[my] = compute_panel(x_l, my)

        rs_r_prev = rs_l_prev = None
        for t in range(N_DEV - 1):
            c_r = lax.rem(my + (N_DEV - t - 1), N_DEV)
            c_l = lax.rem(my + t + 1, N_DEV)
            ag_r.wait_recv()
            ag_l.wait_recv()
            if t < N_DEV - 2:
                ag_r = ag_copy(x_r, c_r, (ag_r_send, ag_r_recv), t + 1, right)
                ag_r.start()
                ag_l = ag_copy(x_l, c_l, (ag_l_send, ag_l_recv), t + 1, left)
                ag_l.start()
                pending += [ag_r, ag_l]
            p_r = compute_panel(x_r, c_r)
            if rs_r_prev is not None:
                rs_r_prev.wait_recv()
                p_r = p_r + rsbuf_r[t - 1]
            acc_r[c_r] = p_r
            rs_r_prev = rs_copy(acc_r, c_r, rsbuf_r,
                                (rs_r_send, rs_r_recv), t, right)
            rs_r_prev.start()
            pending.append(rs_r_prev)

            p_l = compute_panel(x_l, c_l)
            if rs_l_prev is not None:
                rs_l_prev.wait_recv()
                p_l = p_l + rsbuf_l[t - 1]
            acc_l[c_l] = p_l
            rs_l_prev = rs_copy(acc_l, c_l, rsbuf_l,
                                (rs_l_send, rs_l_recv), t, left)
            rs_l_prev.start()
            pending.append(rs_l_prev)

        rs_r_prev.wait_recv()
        out_ref[0] = acc_r[my] + rsbuf_r[N_DEV - 2]
        rs_l_prev.wait_recv()
        out_ref[1] = acc_l[my] + rsbuf_l[N_DEV - 2]

        for rdma in pending:
            rdma.wait_send()

    return pl.pallas_call(
        body,
        out_shape=jax.ShapeDtypeStruct((B_LOC, SQ, D), jnp.float32),
        in_specs=[pl.BlockSpec(memory_space=pltpu.VMEM)] * 5,
        out_specs=pl.BlockSpec(memory_space=pltpu.VMEM),
        scratch_shapes=[
            pltpu.VMEM((N_DEV, SQ, D), jnp.float32),
            pltpu.VMEM((N_DEV, SQ, D), jnp.float32),
            pltpu.VMEM((N_DEV, SQ, D), jnp.float32),
            pltpu.VMEM((N_DEV, SQ, D), jnp.float32),
            pltpu.VMEM((N_DEV - 1, SQ, D), jnp.float32),
            pltpu.VMEM((N_DEV - 1, SQ, D), jnp.float32),
            pltpu.VMEM((SQ, D), jnp.float32),
            pltpu.SemaphoreType.DMA((N_DEV - 1,)),
            pltpu.SemaphoreType.DMA((N_DEV - 1,)),
            pltpu.SemaphoreType.DMA((N_DEV - 1,)),
            pltpu.SemaphoreType.DMA((N_DEV - 1,)),
            pltpu.SemaphoreType.DMA((N_DEV - 1,)),
            pltpu.SemaphoreType.DMA((N_DEV - 1,)),
            pltpu.SemaphoreType.DMA((N_DEV - 1,)),
            pltpu.SemaphoreType.DMA((N_DEV - 1,)),
        ],
        compiler_params=pltpu.CompilerParams(collective_id=0),
    )(x, Wq, Wo, Wk, Wv)


# device time: 65179 ns/iter; 2.3799x vs baseline; 1.0039x over previous
import jax
import jax.numpy as jnp
from jax import lax
from jax.experimental import pallas as pl
from jax.experimental.pallas import tpu as pltpu

N_DEV = 8
B_LOC = 2
SQ = 128
D = 512
H_LOC = 8
DH = 64
SCALE = 0.125


def kernel(x, Wq, Wo, Wk, Wv):
    def body(x_ref, wq_ref, wo_ref, wk_ref, wv_ref, out_ref,
             x_r, x_l, acc_r, acc_l, rsbuf_r, rsbuf_l, o_buf,
             ag_r_send, ag_r_recv, ag_l_send, ag_l_recv,
             rs_r_send, rs_r_recv, rs_l_send, rs_l_recv):
        my = lax.axis_index("i")

        def ring_dev(j):
            return jnp.where(j < 4, j, 11 - j)

        rho = ring_dev(my)
        left = ring_dev(lax.rem(rho + (N_DEV - 1), N_DEV))
        right = ring_dev(lax.rem(rho + 1, N_DEV))

        def compute_panel(x_buf, slot):
            xb = x_buf[slot]
            q = jnp.dot(xb, wq_ref[...], preferred_element_type=jnp.float32)
            k = jnp.dot(xb, wk_ref[...], preferred_element_type=jnp.float32)
            v = jnp.dot(xb, wv_ref[...], preferred_element_type=jnp.float32)
            for hh in range(H_LOC):
                sl = slice(hh * DH, (hh + 1) * DH)
                s = lax.dot_general(
                    q[:, sl], k[:, sl], (((1,), (1,)), ((), ())),
                    preferred_element_type=jnp.float32,
                ) * SCALE
                m = jnp.max(s, axis=1, keepdims=True)
                p = jnp.exp(s - m)
                l = jnp.sum(p, axis=1, keepdims=True)
                o = jnp.dot(p, v[:, sl], preferred_element_type=jnp.float32)
                o_buf[:, sl] = o / l
            return jnp.dot(o_buf[...], wo_ref[...],
                           preferred_element_type=jnp.float32)

        def ag_copy(x_buf, slot, sems, h, dst):
            return pltpu.make_async_remote_copy(
                src_ref=x_buf.at[slot], dst_ref=x_buf.at[slot],
                send_sem=sems[0].at[h], recv_sem=sems[1].at[h],
                device_id=(dst,), device_id_type=pl.DeviceIdType.MESH,
            )

        def rs_copy(acc_buf, slot, rsbuf, sems, t, dst):
            return pltpu.make_async_remote_copy(
                src_ref=acc_buf.at[slot], dst_ref=rsbuf.at[t],
                send_sem=sems[0].at[t], recv_sem=sems[1].at[t],
                device_id=(dst,), device_id_type=pl.DeviceIdType.MESH,
            )

        barrier_sem = pltpu.get_barrier_semaphore()
        pl.semaphore_signal(barrier_sem, inc=1, device_id=(left,),
                            device_id_type=pl.DeviceIdType.MESH)
        pl.semaphore_signal(barrier_sem, inc=1, device_id=(right,),
                            device_id_type=pl.DeviceIdType.MESH)
        pl.semaphore_wait(barrier_sem, 2)

        x_r[my] = x_ref[0]
        x_l[my] = x_ref[1]
        pending = []
        ag_r = ag_copy(x_r, my, (ag_r_send, ag_r_recv), 0, right)
        ag_r.start()
        ag_l = ag_copy(x_l, my, (ag_l_send, ag_l_recv), 0, left)
        ag_l.start()
        pending += [ag_r, ag_l]

        acc_r[my] = compute_panel(x_r, my)
        acc_l[my] = compute_panel(x_l, my)

        rs_r_prev = rs_l_prev = None
        for t in range(N_DEV - 1):
            c_r = ring_dev(lax.rem(rho + (N_DEV - t - 1), N_DEV))
            c_l = ring_dev(lax.rem(rho + t + 1, N_DEV))
            ag_r.wait_recv()
            ag_l.wait_recv()
            if t < N_DEV - 2:
                ag_r = ag_copy(x_r, c_r, (ag_r_send, ag_r_recv), t + 1, right)
                ag_r.start()
                ag_l = ag_copy(x_l, c_l, (ag_l_send, ag_l_recv), t + 1, left)
                ag_l.start()
                pending += [ag_r, ag_l]
            p_r = compute_panel(x_r, c_r)
            if rs_r_prev is not None:
                rs_r_prev.wait_recv()
                p_r = p_r + rsbuf_r[t - 1]
            acc_r[c_r] = p_r
            rs_r_prev = rs_copy(acc_r, c_r, rsbuf_r,
                                (rs_r_send, rs_r_recv), t, right)
            rs_r_prev.start()
            pending.append(rs_r_prev)

            p_l = compute_panel(x_l, c_l)
            if rs_l_prev is not None:
                rs_l_prev.wait_recv()
                p_l = p_l + rsbuf_l[t - 1]
            acc_l[c_l] = p_l
            rs_l_prev = rs_copy(acc_l, c_l, rsbuf_l,
                                (rs_l_send, rs_l_recv), t, left)
            rs_l_prev.start()
            pending.append(rs_l_prev)

        rs_r_prev.wait_recv()
        out_ref[0] = acc_r[my] + rsbuf_r[N_DEV - 2]
        rs_l_prev.wait_recv()
        out_ref[1] = acc_l[my] + rsbuf_l[N_DEV - 2]

        for rdma in pending:
            rdma.wait_send()

    return pl.pallas_call(
        body,
        out_shape=jax.ShapeDtypeStruct((B_LOC, SQ, D), jnp.float32),
        in_specs=[pl.BlockSpec(memory_space=pltpu.VMEM)] * 5,
        out_specs=pl.BlockSpec(memory_space=pltpu.VMEM),
        scratch_shapes=[
            pltpu.VMEM((N_DEV, SQ, D), jnp.float32),
            pltpu.VMEM((N_DEV, SQ, D), jnp.float32),
            pltpu.VMEM((N_DEV, SQ, D), jnp.float32),
            pltpu.VMEM((N_DEV, SQ, D), jnp.float32),
            pltpu.VMEM((N_DEV - 1, SQ, D), jnp.float32),
            pltpu.VMEM((N_DEV - 1, SQ, D), jnp.float32),
            pltpu.VMEM((SQ, D), jnp.float32),
            pltpu.SemaphoreType.DMA((N_DEV - 1,)),
            pltpu.SemaphoreType.DMA((N_DEV - 1,)),
            pltpu.SemaphoreType.DMA((N_DEV - 1,)),
            pltpu.SemaphoreType.DMA((N_DEV - 1,)),
            pltpu.SemaphoreType.DMA((N_DEV - 1,)),
            pltpu.SemaphoreType.DMA((N_DEV - 1,)),
            pltpu.SemaphoreType.DMA((N_DEV - 1,)),
            pltpu.SemaphoreType.DMA((N_DEV - 1,)),
        ],
        compiler_params=pltpu.CompilerParams(collective_id=0),
    )(x, Wq, Wo, Wk, Wv)


# device time: 62083 ns/iter; 2.4986x vs baseline; 1.0499x over previous
import jax
import jax.numpy as jnp
from jax import lax
from jax.experimental import pallas as pl
from jax.experimental.pallas import tpu as pltpu

N_DEV = 8
B_LOC = 2
SQ = 128
D = 512
H_LOC = 8
DH = 64
SCALE = 0.125


def kernel(x, Wq, Wo, Wk, Wv):
    def body(x_ref, wq_ref, wo_ref, wk_ref, wv_ref, out_ref,
             x_all, d_send, d_recv, acc, o_buf,
             wq_bf, wk_bf, wv_bf, wo_bf,
             ag_send, ag_recv, rs_send, rs_recv):
        my = lax.axis_index("i")

        barrier_sem = pltpu.get_barrier_semaphore()
        for k in range(1, N_DEV):
            pl.semaphore_signal(
                barrier_sem, inc=1,
                device_id=(lax.rem(my + k, N_DEV),),
                device_id_type=pl.DeviceIdType.MESH,
            )
        pl.semaphore_wait(barrier_sem, N_DEV - 1)

        x_all[my] = x_ref[...].astype(jnp.bfloat16)
        pending = []
        for k in range(1, N_DEV):
            rdma = pltpu.make_async_remote_copy(
                src_ref=x_all.at[my], dst_ref=x_all.at[my],
                send_sem=ag_send.at[k - 1], recv_sem=ag_recv.at[my],
                device_id=(lax.rem(my + k, N_DEV),),
                device_id_type=pl.DeviceIdType.MESH,
            )
            rdma.start()
            pending.append(rdma)

        wq_bf[...] = wq_ref[...].astype(jnp.bfloat16)
        wk_bf[...] = wk_ref[...].astype(jnp.bfloat16)
        wv_bf[...] = wv_ref[...].astype(jnp.bfloat16)
        wo_bf[...] = wo_ref[...].astype(jnp.bfloat16)

        def compute_chunk(slot, b):
            xb = x_all[slot, b]
            q = jnp.dot(xb, wq_bf[...], preferred_element_type=jnp.float32)
            k = jnp.dot(xb, wk_bf[...], preferred_element_type=jnp.float32)
            v = jnp.dot(xb, wv_bf[...], preferred_element_type=jnp.float32)
            vb = v.astype(jnp.bfloat16)
            for hh in range(H_LOC):
                sl = slice(hh * DH, (hh + 1) * DH)
                s = lax.dot_general(
                    q[:, sl], k[:, sl], (((1,), (1,)), ((), ())),
                    preferred_element_type=jnp.float32,
                ) * SCALE
                m = jnp.max(s, axis=1, keepdims=True)
                p = jnp.exp(s - m)
                l = jnp.sum(p, axis=1, keepdims=True)
                o = jnp.dot(p.astype(jnp.bfloat16), vb[:, sl],
                            preferred_element_type=jnp.float32)
                o_buf[:, sl] = o / l
            return jnp.dot(o_buf[...].astype(jnp.bfloat16), wo_bf[...],
                           preferred_element_type=jnp.float32)

        for b in range(B_LOC):
            acc[b] = compute_chunk(my, b)

        for k in range(1, N_DEV):
            src = lax.rem(my + (N_DEV - k), N_DEV)
            pltpu.make_async_remote_copy(
                src_ref=x_all.at[src], dst_ref=x_all.at[src],
                send_sem=ag_send.at[k - 1], recv_sem=ag_recv.at[src],
                device_id=(src,), device_id_type=pl.DeviceIdType.MESH,
            ).wait_recv()
            for b in range(B_LOC):
                d_send[k - 1, b] = compute_chunk(src, b).astype(jnp.bfloat16)
            rdma = pltpu.make_async_remote_copy(
                src_ref=d_send.at[k - 1], dst_ref=d_recv.at[my],
                send_sem=rs_send.at[k - 1], recv_sem=rs_recv.at[my],
                device_id=(src,),
                device_id_type=pl.DeviceIdType.MESH,
            )
            rdma.start()
            pending.append(rdma)

        for k in range(1, N_DEV):
            src = lax.rem(my + k, N_DEV)
            pltpu.make_async_remote_copy(
                src_ref=d_recv.at[src], dst_ref=d_recv.at[src],
                send_sem=rs_send.at[k - 1], recv_sem=rs_recv.at[src],
                device_id=(src,), device_id_type=pl.DeviceIdType.MESH,
            ).wait_recv()
            for b in range(B_LOC):
                acc[b] = acc[b] + d_recv[src, b].astype(jnp.float32)

        out_ref[...] = acc[...]

        for rdma in pending:
            rdma.wait_send()

    return pl.pallas_call(
        body,
        out_shape=jax.ShapeDtypeStruct((B_LOC, SQ, D), jnp.float32),
        in_specs=[pl.BlockSpec(memory_space=pltpu.VMEM)] * 5,
        out_specs=pl.BlockSpec(memory_space=pltpu.VMEM),
        scratch_shapes=[
            pltpu.VMEM((N_DEV, B_LOC, SQ, D), jnp.bfloat16),
            pltpu.VMEM((N_DEV - 1, B_LOC, SQ, D), jnp.bfloat16),
            pltpu.VMEM((N_DEV, B_LOC, SQ, D), jnp.bfloat16),
            pltpu.VMEM((B_LOC, SQ, D), jnp.float32),
            pltpu.VMEM((SQ, D), jnp.float32),
            pltpu.VMEM((D, H_LOC * DH), jnp.bfloat16),
            pltpu.VMEM((D, H_LOC * DH), jnp.bfloat16),
            pltpu.VMEM((D, H_LOC * DH), jnp.bfloat16),
            pltpu.VMEM((H_LOC * DH, D), jnp.bfloat16),
            pltpu.SemaphoreType.DMA((N_DEV - 1,)),
            pltpu.SemaphoreType.DMA((N_DEV,)),
            pltpu.SemaphoreType.DMA((N_DEV - 1,)),
            pltpu.SemaphoreType.DMA((N_DEV,)),
        ],
        compiler_params=pltpu.CompilerParams(collective_id=0),
    )(x, Wq, Wo, Wk, Wv)


# device time: 46136 ns/iter; 3.3622x vs baseline; 1.3457x over previous
import jax
import jax.numpy as jnp
from jax import lax
from jax.experimental import pallas as pl
from jax.experimental.pallas import tpu as pltpu

N_DEV = 8
B_LOC = 2
SQ = 128
D = 512
H_LOC = 8
DH = 64
SCALE = 0.125


def kernel(x, Wq, Wo, Wk, Wv):
    def body(x_ref, wq_ref, wo_ref, wk_ref, wv_ref, out_ref,
             x_all, d_send, d_recv, acc, wqkv_bf, wo_bf,
             ag_send, ag_recv, rs_send, rs_recv):
        my = lax.axis_index("i")

        barrier_sem = pltpu.get_barrier_semaphore()
        for k in range(1, N_DEV):
            pl.semaphore_signal(
                barrier_sem, inc=1,
                device_id=(lax.rem(my + k, N_DEV),),
                device_id_type=pl.DeviceIdType.MESH,
            )
        pl.semaphore_wait(barrier_sem, N_DEV - 1)

        for b in range(B_LOC):
            x_all[my, b] = x_ref[b].T.astype(jnp.bfloat16)
        pending = []
        for k in range(1, N_DEV):
            rdma = pltpu.make_async_remote_copy(
                src_ref=x_all.at[my], dst_ref=x_all.at[my],
                send_sem=ag_send.at[k - 1], recv_sem=ag_recv.at[my],
                device_id=(lax.rem(my + k, N_DEV),),
                device_id_type=pl.DeviceIdType.MESH,
            )
            rdma.start()
            pending.append(rdma)

        wqkv_bf[:, 0:512] = wq_ref[...].astype(jnp.bfloat16)
        wqkv_bf[:, 512:1024] = wk_ref[...].astype(jnp.bfloat16)
        wqkv_bf[:, 1024:1536] = wv_ref[...].astype(jnp.bfloat16)
        wo_bf[...] = wo_ref[...].astype(jnp.bfloat16)

        def compute_chunk(slot, b):
            xt = x_all[slot, b]
            qkvt = lax.dot_general(
                wqkv_bf[...], xt, (((0,), (0,)), ((), ())),
                preferred_element_type=jnp.float32,
            )
            qt = qkvt[0:512].reshape(H_LOC, DH, SQ).astype(jnp.bfloat16)
            kt = qkvt[512:1024].reshape(H_LOC, DH, SQ).astype(jnp.bfloat16)
            vt = qkvt[1024:1536].reshape(H_LOC, DH, SQ).astype(jnp.bfloat16)
            s = lax.dot_general(
                qt, kt, (((1,), (1,)), ((0,), (0,))),
                preferred_element_type=jnp.float32,
            ) * SCALE
            m = jnp.max(s, axis=2, keepdims=True)
            p = jnp.exp(s - m)
            l = jnp.sum(p, axis=2)
            ot = lax.dot_general(
                vt, p.astype(jnp.bfloat16), (((2,), (2,)), ((0,), (0,))),
                preferred_element_type=jnp.float32,
            )
            ot = ot / l[:, None, :]
            return lax.dot_general(
                wo_bf[...], ot.astype(jnp.bfloat16).reshape(H_LOC * DH, SQ),
                (((0,), (0,)), ((), ())),
                preferred_element_type=jnp.float32,
            )

        for b in range(B_LOC):
            acc[b] = compute_chunk(my, b)

        for k in range(1, N_DEV):
            src = lax.rem(my + (N_DEV - k), N_DEV)
            pltpu.make_async_remote_copy(
                src_ref=x_all.at[src], dst_ref=x_all.at[src],
                send_sem=ag_send.at[k - 1], recv_sem=ag_recv.at[src],
                device_id=(src,), device_id_type=pl.DeviceIdType.MESH,
            ).wait_recv()
            for b in range(B_LOC):
                d_send[k - 1, b] = compute_chunk(src, b).astype(jnp.bfloat16)
            rdma = pltpu.make_async_remote_copy(
                src_ref=d_send.at[k - 1], dst_ref=d_recv.at[my],
                send_sem=rs_send.at[k - 1], recv_sem=rs_recv.at[my],
                device_id=(src,),
                device_id_type=pl.DeviceIdType.MESH,
            )
            rdma.start()
            pending.append(rdma)

        for k in range(1, N_DEV):
            src = lax.rem(my + k, N_DEV)
            pltpu.make_async_remote_copy(
                src_ref=d_recv.at[src], dst_ref=d_recv.at[src],
                send_sem=rs_send.at[k - 1], recv_sem=rs_recv.at[src],
                device_id=(src,), device_id_type=pl.DeviceIdType.MESH,
            ).wait_recv()
            for b in range(B_LOC):
                acc[b] = acc[b] + d_recv[src, b].astype(jnp.float32)

        for b in range(B_LOC):
            out_ref[b] = acc[b].T

        for rdma in pending:
            rdma.wait_send()

    return pl.pallas_call(
        body,
        out_shape=jax.ShapeDtypeStruct((B_LOC, SQ, D), jnp.float32),
        in_specs=[pl.BlockSpec(memory_space=pltpu.VMEM)] * 5,
        out_specs=pl.BlockSpec(memory_space=pltpu.VMEM),
        scratch_shapes=[
            pltpu.VMEM((N_DEV, B_LOC, D, SQ), jnp.bfloat16),
            pltpu.VMEM((N_DEV - 1, B_LOC, D, SQ), jnp.bfloat16),
            pltpu.VMEM((N_DEV, B_LOC, D, SQ), jnp.bfloat16),
            pltpu.VMEM((B_LOC, D, SQ), jnp.float32),
            pltpu.VMEM((D, 3 * H_LOC * DH), jnp.bfloat16),
            pltpu.VMEM((H_LOC * DH, D), jnp.bfloat16),
            pltpu.SemaphoreType.DMA((N_DEV - 1,)),
            pltpu.SemaphoreType.DMA((N_DEV,)),
            pltpu.SemaphoreType.DMA((N_DEV - 1,)),
            pltpu.SemaphoreType.DMA((N_DEV,)),
        ],
        compiler_params=pltpu.CompilerParams(collective_id=0),
    )(x, Wq, Wo, Wk, Wv)
